# baseline (speedup 1.0000x reference)
import functools

import jax
import jax.numpy as jnp
from jax import lax
from jax.experimental import pallas as pl
from jax.experimental.pallas import tpu as pltpu
from jax.experimental.pallas import tpu_sc as plsc

B, N, D, K, E = 4, 4096, 128, 27, 256
NT = 256
T = N // NT
SG = 16
KP = 32


S = 6
NC_ = N // 128


def _topk_kernel(xv_ref, xvt_ref, out_ref, dmat_ref, flags_ref):
    b = pl.program_id(0)
    xq = xv_ref[0]
    xpt = xvt_ref[0]
    d2q = (xq[:, 0:1] * xq[:, 0:1] + xq[:, 1:2] * xq[:, 1:2]) + xq[:, 2:3] * xq[:, 2:3]
    d2p = (xpt[0:1, :] * xpt[0:1, :] + xpt[1:2, :] * xpt[1:2, :]) + xpt[2:3, :] * xpt[2:3, :]
    dot = jnp.dot(xq, xpt, preferred_element_type=jnp.float32)
    dmat_ref[...] = (d2q + d2p) - 2.0 * dot

    base = b * N
    inf = jnp.float32(jnp.inf)
    bigi = jnp.int32(N)
    liota = lax.broadcasted_iota(jnp.int32, (SG, 128), 1)
    kiota = lax.broadcasted_iota(jnp.int32, (SG, KP), 1)
    kiota128 = liota

    def subgroup(rg, _):
        v = [jnp.full((SG, 128), inf, jnp.float32) for _ in range(S)]
        gi = [jnp.full((SG, 128), bigi, jnp.int32) for _ in range(S)]
        for c in range(NC_):
            t = dmat_ref[pl.ds(rg * SG, SG), pl.ds(c * 128, 128)]
            ti = liota + (c * 128)
            for j in range(S):
                lt = t < v[j]
                nv = jnp.where(lt, t, v[j])
                nt = jnp.where(lt, v[j], t)
                ni = jnp.where(lt, ti, gi[j])
                nti = jnp.where(lt, gi[j], ti)
                v[j], gi[j], t, ti = nv, ni, nt, nti
        res = jnp.zeros((SG, 128), jnp.int32)
        flag = jnp.zeros((SG, 128), jnp.int32)
        for k in range(K):
            m = v[0]
            for sh in (1, 2, 4, 8, 16, 32, 64):
                m = jnp.minimum(m, pltpu.roll(m, sh, 1))
            ik = jnp.where(v[0] == m, gi[0], bigi)
            for sh in (1, 2, 4, 8, 16, 32, 64):
                ik = jnp.minimum(ik, pltpu.roll(ik, sh, 1))
            res = jnp.where(kiota128 == k, ik + base, res)
            ext = (v[0] == m) & (gi[0] == ik)
            for j in range(S - 1):
                v[j] = jnp.where(ext, v[j + 1], v[j])
                gi[j] = jnp.where(ext, gi[j + 1], gi[j])
            v[S - 1] = jnp.where(ext, inf, v[S - 1])
            gi[S - 1] = jnp.where(ext, bigi, gi[S - 1])
            flag = flag | jnp.where(ext & (v[0] == inf), 1, 0)
        out_ref[0, 0, pl.ds(rg * SG, SG), :] = res[:, :KP]
        flags_ref[rg] = jnp.max(flag)
        return 0

    lax.fori_loop(0, NT // SG, subgroup, 0)

    giota = lax.broadcasted_iota(jnp.int32, (SG, N), 1)

    def fb_subgroup(rg, _):
        @pl.when(flags_ref[rg] > 0)
        def _():
            def fb_k(k, res):
                d = dmat_ref[pl.ds(rg * SG, SG), :]
                m = jnp.min(d, axis=1, keepdims=True)
                ik = jnp.min(jnp.where(d == m, giota, bigi), axis=1, keepdims=True)
                dmat_ref[pl.ds(rg * SG, SG), :] = jnp.where(giota == ik, inf, d)
                return jnp.where(kiota == k, ik + base, res)

            res = lax.fori_loop(0, K, fb_k, jnp.zeros((SG, KP), jnp.int32))
            out_ref[0, 0, pl.ds(rg * SG, SG), :] = res

        return 0

    lax.fori_loop(0, NT // SG, fb_subgroup, 0)


def _run_topk(x_v):
    xvt = x_v.transpose(0, 2, 1)
    return pl.pallas_call(
        _topk_kernel,
        grid=(B, T),
        in_specs=[
            pl.BlockSpec((1, NT, 3), lambda b, t: (b, t, 0)),
            pl.BlockSpec((1, 3, N), lambda b, t: (b, 0, 0)),
        ],
        out_specs=pl.BlockSpec((1, 1, NT, KP), lambda b, t: (b, t, 0, 0)),
        out_shape=jax.ShapeDtypeStruct((B, T, NT, KP), jnp.int32),
        scratch_shapes=[
            pltpu.VMEM((NT, N), jnp.float32),
            pltpu.SMEM((NT // SG,), jnp.int32),
        ],
    )(x_v, xvt)


NW = 32
ROWS = B * N * K
PW = ROWS // NW
CH = 512
NCH = PW // CH


def _gather_body(table_hbm, idx_hbm, out_hbm, idx_v, rows_v, sem):
    wid = lax.axis_index("s") * 2 + lax.axis_index("c")
    base = wid * PW
    pltpu.sync_copy(idx_hbm.at[pl.ds(base, PW)], idx_v)

    def chunk(ci, _):
        pltpu.async_copy(
            table_hbm.at[idx_v.at[pl.ds(ci * CH, CH)]], rows_v, sem
        ).wait()
        pltpu.sync_copy(rows_v, out_hbm.at[pl.ds(base + ci * CH, CH)])
        return 0

    lax.fori_loop(0, NCH, chunk, 0)


@functools.cache
def _make_gather():
    return pl.kernel(
        _gather_body,
        mesh=plsc.VectorSubcoreMesh(core_axis_name="c", subcore_axis_name="s"),
        out_type=jax.ShapeDtypeStruct((ROWS, D), jnp.float32),
        scratch_types=[
            pltpu.VMEM((PW,), jnp.int32),
            pltpu.VMEM((CH, D), jnp.float32),
            pltpu.SemaphoreType.DMA,
        ],
    )


def _mm_kernel(flat_ref, wt_ref, b_ref, out_ref):
    out_ref[...] = (
        jnp.dot(flat_ref[...], wt_ref[...], preferred_element_type=jnp.float32)
        + b_ref[...]
    )


def _run_mm(flat, wt, bias):
    return pl.pallas_call(
        _mm_kernel,
        grid=(B * T,),
        in_specs=[
            pl.BlockSpec((NT, K * D), lambda i: (i, 0)),
            pl.BlockSpec((K * D, E), lambda i: (0, 0)),
            pl.BlockSpec((1, E), lambda i: (0, 0)),
        ],
        out_specs=pl.BlockSpec((NT, E), lambda i: (i, 0)),
        out_shape=jax.ShapeDtypeStruct((B * T * NT, E), jnp.float32),
    )(flat, wt, bias)


def kernel(x, x_v, W, b):
    idxg = _run_topk(x_v)
    idx = idxg[..., :K].reshape(ROWS)
    table = x.reshape(B * N, D)
    g = _make_gather()(table, idx)
    flat = g.reshape(B * T * NT, K * D)
    out = _run_mm(flat, W.T, b.reshape(1, E))
    return out.reshape(B, N, E)


# speedup vs baseline: 6.7033x; 6.7033x over previous
import functools

import jax
import jax.numpy as jnp
from jax import lax
from jax.experimental import pallas as pl
from jax.experimental.pallas import tpu as pltpu
from jax.experimental.pallas import tpu_sc as plsc

B, N, D, K, E = 4, 4096, 128, 27, 256
NT = 256
T = N // NT
SG = 16
KP = 32


S = 6
NC_ = N // 128


def _topk_kernel(xv_ref, xvt_ref, out_ref, dmat_ref, flags_ref):
    b = pl.program_id(0)
    xq = xv_ref[0]
    xpt = xvt_ref[0]
    d2q = (xq[:, 0:1] * xq[:, 0:1] + xq[:, 1:2] * xq[:, 1:2]) + xq[:, 2:3] * xq[:, 2:3]
    d2p = (xpt[0:1, :] * xpt[0:1, :] + xpt[1:2, :] * xpt[1:2, :]) + xpt[2:3, :] * xpt[2:3, :]
    dot = jnp.dot(xq, xpt, preferred_element_type=jnp.float32)
    dmat_ref[...] = (d2q + d2p) - 2.0 * dot

    base = b * N
    inf = jnp.float32(jnp.inf)
    bigi = jnp.int32(N)
    liota = lax.broadcasted_iota(jnp.int32, (SG, 128), 1)
    kiota = lax.broadcasted_iota(jnp.int32, (SG, KP), 1)
    kiota128 = liota

    def subgroup(rg):
        v = [jnp.full((SG, 128), inf, jnp.float32) for _ in range(S)]
        gi = [jnp.full((SG, 128), bigi, jnp.int32) for _ in range(S)]
        for c in range(NC_):
            t = dmat_ref[pl.ds(rg * SG, SG), pl.ds(c * 128, 128)]
            ti = liota + (c * 128)
            for j in range(S):
                lt = t < v[j]
                nv = jnp.where(lt, t, v[j])
                nt = jnp.where(lt, v[j], t)
                ni = jnp.where(lt, ti, gi[j])
                nti = jnp.where(lt, gi[j], ti)
                v[j], gi[j], t, ti = nv, ni, nt, nti
        res = jnp.zeros((SG, KP), jnp.int32)
        flag = jnp.zeros((SG, 128), jnp.int32)
        for k in range(K):
            m = jnp.min(v[0], axis=1, keepdims=True)
            ik = jnp.min(jnp.where(v[0] == m, gi[0], bigi), axis=1, keepdims=True)
            res = jnp.where(kiota == k, ik + base, res)
            ext = (v[0] == m) & (gi[0] == ik)
            for j in range(S - 1):
                v[j] = jnp.where(ext, v[j + 1], v[j])
                gi[j] = jnp.where(ext, gi[j + 1], gi[j])
            v[S - 1] = jnp.where(ext, inf, v[S - 1])
            gi[S - 1] = jnp.where(ext, bigi, gi[S - 1])
            flag = flag | jnp.where(ext & (v[0] == inf), 1, 0)
        out_ref[0, 0, pl.ds(rg * SG, SG), :] = res
        flags_ref[rg] = jnp.max(flag)

    def pair(i, _):
        subgroup(2 * i)
        subgroup(2 * i + 1)
        return 0

    lax.fori_loop(0, NT // SG // 2, pair, 0)

    giota = lax.broadcasted_iota(jnp.int32, (SG, N), 1)

    def fb_subgroup(rg, _):
        @pl.when(flags_ref[rg] > 0)
        def _():
            def fb_k(k, res):
                d = dmat_ref[pl.ds(rg * SG, SG), :]
                m = jnp.min(d, axis=1, keepdims=True)
                ik = jnp.min(jnp.where(d == m, giota, bigi), axis=1, keepdims=True)
                dmat_ref[pl.ds(rg * SG, SG), :] = jnp.where(giota == ik, inf, d)
                return jnp.where(kiota == k, ik + base, res)

            res = lax.fori_loop(0, K, fb_k, jnp.zeros((SG, KP), jnp.int32))
            out_ref[0, 0, pl.ds(rg * SG, SG), :] = res

        return 0

    lax.fori_loop(0, NT // SG, fb_subgroup, 0)


def _run_topk(x_v):
    xvt = x_v.transpose(0, 2, 1)
    return pl.pallas_call(
        _topk_kernel,
        grid=(B, T),
        in_specs=[
            pl.BlockSpec((1, NT, 3), lambda b, t: (b, t, 0)),
            pl.BlockSpec((1, 3, N), lambda b, t: (b, 0, 0)),
        ],
        out_specs=pl.BlockSpec((1, 1, NT, KP), lambda b, t: (b, t, 0, 0)),
        out_shape=jax.ShapeDtypeStruct((B, T, NT, KP), jnp.int32),
        scratch_shapes=[
            pltpu.VMEM((NT, N), jnp.float32),
            pltpu.SMEM((NT // SG,), jnp.int32),
        ],
    )(x_v, xvt)


NW = 32
ROWS = B * N * K
PW = ROWS // NW
CH = 512
NCH = PW // CH


def _gather_body(table_hbm, idx_hbm, out_hbm, idx_v, rows_v, sem):
    wid = lax.axis_index("s") * 2 + lax.axis_index("c")
    base = wid * PW
    pltpu.sync_copy(idx_hbm.at[pl.ds(base, PW)], idx_v)

    def chunk(ci, _):
        pltpu.async_copy(
            table_hbm.at[idx_v.at[pl.ds(ci * CH, CH)]], rows_v, sem
        ).wait()
        pltpu.sync_copy(rows_v, out_hbm.at[pl.ds(base + ci * CH, CH)])
        return 0

    lax.fori_loop(0, NCH, chunk, 0)


@functools.cache
def _make_gather():
    return pl.kernel(
        _gather_body,
        mesh=plsc.VectorSubcoreMesh(core_axis_name="c", subcore_axis_name="s"),
        out_type=jax.ShapeDtypeStruct((ROWS, D), jnp.float32),
        scratch_types=[
            pltpu.VMEM((PW,), jnp.int32),
            pltpu.VMEM((CH, D), jnp.float32),
            pltpu.SemaphoreType.DMA,
        ],
    )


def _mm_kernel(flat_ref, wt_ref, b_ref, out_ref):
    out_ref[...] = (
        jnp.dot(flat_ref[...], wt_ref[...], preferred_element_type=jnp.float32)
        + b_ref[...]
    )


def _run_mm(flat, wt, bias):
    return pl.pallas_call(
        _mm_kernel,
        grid=(B * T,),
        in_specs=[
            pl.BlockSpec((NT, K * D), lambda i: (i, 0)),
            pl.BlockSpec((K * D, E), lambda i: (0, 0)),
            pl.BlockSpec((1, E), lambda i: (0, 0)),
        ],
        out_specs=pl.BlockSpec((NT, E), lambda i: (i, 0)),
        out_shape=jax.ShapeDtypeStruct((B * T * NT, E), jnp.float32),
    )(flat, wt, bias)


def kernel(x, x_v, W, b):
    idxg = _run_topk(x_v)
    idx = idxg[..., :K].reshape(ROWS)
    table = x.reshape(B * N, D)
    g = _make_gather()(table, idx)
    flat = g.reshape(B * T * NT, K * D)
    out = _run_mm(flat, W.T, b.reshape(1, E))
    return out.reshape(B, N, E)


# speedup vs baseline: 10.5920x; 1.5801x over previous
import functools

import jax
import jax.numpy as jnp
from jax import lax
from jax.experimental import pallas as pl
from jax.experimental.pallas import tpu as pltpu
from jax.experimental.pallas import tpu_sc as plsc

B, N, D, K, E = 4, 4096, 128, 27, 256
NT = 256
T = N // NT
SG = 16
KP = 32


S = 6
NC_ = N // 128


def _topk_kernel(xv_ref, xvt_ref, out_ref, dmat_ref, flags_ref):
    b = pl.program_id(0)
    xq = xv_ref[0]
    xpt = xvt_ref[0]
    d2q = (xq[:, 0:1] * xq[:, 0:1] + xq[:, 1:2] * xq[:, 1:2]) + xq[:, 2:3] * xq[:, 2:3]
    d2p = (xpt[0:1, :] * xpt[0:1, :] + xpt[1:2, :] * xpt[1:2, :]) + xpt[2:3, :] * xpt[2:3, :]
    dot = jnp.dot(xq, xpt, preferred_element_type=jnp.float32)
    dmat_ref[...] = (d2q + d2p) - 2.0 * dot

    base = b * N
    inf = jnp.float32(jnp.inf)
    bigi = jnp.int32(N)
    liota = lax.broadcasted_iota(jnp.int32, (SG, 128), 1)
    kiota = lax.broadcasted_iota(jnp.int32, (SG, KP), 1)
    kiota128 = liota

    def subgroup(rg):
        v = [jnp.full((SG, 128), inf, jnp.float32) for _ in range(S)]
        gi = [jnp.full((SG, 128), bigi, jnp.int32) for _ in range(S)]
        for c in range(NC_):
            t = dmat_ref[pl.ds(rg * SG, SG), pl.ds(c * 128, 128)]
            ti = liota + (c * 128)
            for j in range(S):
                lt = t < v[j]
                nv = jnp.where(lt, t, v[j])
                nt = jnp.where(lt, v[j], t)
                ni = jnp.where(lt, ti, gi[j])
                nti = jnp.where(lt, gi[j], ti)
                v[j], gi[j], t, ti = nv, ni, nt, nti
        res = jnp.zeros((SG, KP), jnp.int32)
        flag = jnp.zeros((SG, 128), jnp.int32)
        for k in range(K):
            m = jnp.min(v[0], axis=1, keepdims=True)
            ik = jnp.min(jnp.where(v[0] == m, gi[0], bigi), axis=1, keepdims=True)
            res = jnp.where(kiota == k, ik + base, res)
            ext = (v[0] == m) & (gi[0] == ik)
            for j in range(S - 1):
                v[j] = jnp.where(ext, v[j + 1], v[j])
                gi[j] = jnp.where(ext, gi[j + 1], gi[j])
            v[S - 1] = jnp.where(ext, inf, v[S - 1])
            gi[S - 1] = jnp.where(ext, bigi, gi[S - 1])
            flag = flag | jnp.where(ext & (v[0] == inf), 1, 0)
        out_ref[0, 0, pl.ds(rg * SG, SG), :] = res
        flags_ref[rg] = jnp.max(flag)

    def pair(i, _):
        subgroup(4 * i)
        subgroup(4 * i + 1)
        subgroup(4 * i + 2)
        subgroup(4 * i + 3)
        return 0

    lax.fori_loop(0, NT // SG // 4, pair, 0)

    giota = lax.broadcasted_iota(jnp.int32, (SG, N), 1)

    def fb_subgroup(rg, _):
        @pl.when(flags_ref[rg] > 0)
        def _():
            def fb_k(k, res):
                d = dmat_ref[pl.ds(rg * SG, SG), :]
                m = jnp.min(d, axis=1, keepdims=True)
                ik = jnp.min(jnp.where(d == m, giota, bigi), axis=1, keepdims=True)
                dmat_ref[pl.ds(rg * SG, SG), :] = jnp.where(giota == ik, inf, d)
                return jnp.where(kiota == k, ik + base, res)

            res = lax.fori_loop(0, K, fb_k, jnp.zeros((SG, KP), jnp.int32))
            out_ref[0, 0, pl.ds(rg * SG, SG), :] = res

        return 0

    lax.fori_loop(0, NT // SG, fb_subgroup, 0)


def _run_topk(x_v):
    xvt = x_v.transpose(0, 2, 1)
    return pl.pallas_call(
        _topk_kernel,
        grid=(B, T),
        in_specs=[
            pl.BlockSpec((1, NT, 3), lambda b, t: (b, t, 0)),
            pl.BlockSpec((1, 3, N), lambda b, t: (b, 0, 0)),
        ],
        out_specs=pl.BlockSpec((1, 1, NT, KP), lambda b, t: (b, t, 0, 0)),
        out_shape=jax.ShapeDtypeStruct((B, T, NT, KP), jnp.int32),
        scratch_shapes=[
            pltpu.VMEM((NT, N), jnp.float32),
            pltpu.SMEM((NT // SG,), jnp.int32),
        ],
    )(x_v, xvt)


NW = 32
ROWS = B * N * K
PW = ROWS // NW
CH = 512
NCH = PW // CH


def _gather_body(table_hbm, idx_hbm, out_hbm, idx_v, rows_v, sem):
    wid = lax.axis_index("s") * 2 + lax.axis_index("c")
    base = wid * PW
    pltpu.sync_copy(idx_hbm.at[pl.ds(base, PW)], idx_v)

    def chunk(ci, _):
        pltpu.async_copy(
            table_hbm.at[idx_v.at[pl.ds(ci * CH, CH)]], rows_v, sem
        ).wait()
        pltpu.sync_copy(rows_v, out_hbm.at[pl.ds(base + ci * CH, CH)])
        return 0

    lax.fori_loop(0, NCH, chunk, 0)


@functools.cache
def _make_gather():
    return pl.kernel(
        _gather_body,
        mesh=plsc.VectorSubcoreMesh(core_axis_name="c", subcore_axis_name="s"),
        out_type=jax.ShapeDtypeStruct((ROWS, D), jnp.float32),
        scratch_types=[
            pltpu.VMEM((PW,), jnp.int32),
            pltpu.VMEM((CH, D), jnp.float32),
            pltpu.SemaphoreType.DMA,
        ],
    )


def _mm_kernel(flat_ref, wt_ref, b_ref, out_ref):
    out_ref[...] = (
        jnp.dot(flat_ref[...], wt_ref[...], preferred_element_type=jnp.float32)
        + b_ref[...]
    )


def _run_mm(flat, wt, bias):
    return pl.pallas_call(
        _mm_kernel,
        grid=(B * T,),
        in_specs=[
            pl.BlockSpec((NT, K * D), lambda i: (i, 0)),
            pl.BlockSpec((K * D, E), lambda i: (0, 0)),
            pl.BlockSpec((1, E), lambda i: (0, 0)),
        ],
        out_specs=pl.BlockSpec((NT, E), lambda i: (i, 0)),
        out_shape=jax.ShapeDtypeStruct((B * T * NT, E), jnp.float32),
    )(flat, wt, bias)


def kernel(x, x_v, W, b):
    idxg = _run_topk(x_v)
    idx = idxg[..., :K].reshape(ROWS)
    table = x.reshape(B * N, D)
    g = _make_gather()(table, idx)
    flat = g.reshape(B * T * NT, K * D)
    out = _run_mm(flat, W.T, b.reshape(1, E))
    return out.reshape(B, N, E)


# speedup vs baseline: 14.0910x; 1.3303x over previous
import functools

import jax
import jax.numpy as jnp
from jax import lax
from jax.experimental import pallas as pl
from jax.experimental.pallas import tpu as pltpu
from jax.experimental.pallas import tpu_sc as plsc

B, N, D, K, E = 4, 4096, 128, 27, 256
NT = 256
T = N // NT
SG = 16
KP = 32


S = 6
NC_ = N // 128


def _topk_kernel(xv_ref, xvt_ref, out_ref, dmat_ref, flags_ref):
    b = pl.program_id(0)
    xq = xv_ref[0]
    xpt = xvt_ref[0]
    d2q = (xq[:, 0:1] * xq[:, 0:1] + xq[:, 1:2] * xq[:, 1:2]) + xq[:, 2:3] * xq[:, 2:3]
    d2p = (xpt[0:1, :] * xpt[0:1, :] + xpt[1:2, :] * xpt[1:2, :]) + xpt[2:3, :] * xpt[2:3, :]
    dot = jnp.dot(xq, xpt, preferred_element_type=jnp.float32)
    dmat_ref[...] = (d2q + d2p) - 2.0 * dot

    base = b * N
    inf = jnp.float32(jnp.inf)
    bigi = jnp.int32(N)
    liota = lax.broadcasted_iota(jnp.int32, (SG, 128), 1)
    kiota = lax.broadcasted_iota(jnp.int32, (SG, KP), 1)
    kiota128 = liota

    def subgroup(rg):
        v = [jnp.full((SG, 128), inf, jnp.float32) for _ in range(S)]
        gi = [jnp.full((SG, 128), bigi, jnp.int32) for _ in range(S)]
        for c in range(NC_):
            t = dmat_ref[pl.ds(rg * SG, SG), pl.ds(c * 128, 128)]
            ti = liota + (c * 128)
            for j in range(S):
                lt = t < v[j]
                nv = jnp.where(lt, t, v[j])
                nt = jnp.where(lt, v[j], t)
                ni = jnp.where(lt, ti, gi[j])
                nti = jnp.where(lt, gi[j], ti)
                v[j], gi[j], t, ti = nv, ni, nt, nti
        res = jnp.zeros((SG, KP), jnp.int32)
        flag = jnp.zeros((SG, 128), jnp.int32)
        for k in range(K):
            m = jnp.min(v[0], axis=1, keepdims=True)
            ik = jnp.min(jnp.where(v[0] == m, gi[0], bigi), axis=1, keepdims=True)
            res = jnp.where(kiota == k, ik + base, res)
            ext = (v[0] == m) & (gi[0] == ik)
            for j in range(S - 1):
                v[j] = jnp.where(ext, v[j + 1], v[j])
                gi[j] = jnp.where(ext, gi[j + 1], gi[j])
            v[S - 1] = jnp.where(ext, inf, v[S - 1])
            gi[S - 1] = jnp.where(ext, bigi, gi[S - 1])
            flag = flag | jnp.where(ext & (v[0] == inf), 1, 0)
        out_ref[0, 0, pl.ds(rg * SG, SG), :] = res
        flags_ref[rg] = jnp.max(flag)

    def pair(i, _):
        for j in range(8):
            subgroup(8 * i + j)
        return 0

    lax.fori_loop(0, NT // SG // 8, pair, 0)

    giota = lax.broadcasted_iota(jnp.int32, (SG, N), 1)

    def fb_subgroup(rg, _):
        @pl.when(flags_ref[rg] > 0)
        def _():
            def fb_k(k, res):
                d = dmat_ref[pl.ds(rg * SG, SG), :]
                m = jnp.min(d, axis=1, keepdims=True)
                ik = jnp.min(jnp.where(d == m, giota, bigi), axis=1, keepdims=True)
                dmat_ref[pl.ds(rg * SG, SG), :] = jnp.where(giota == ik, inf, d)
                return jnp.where(kiota == k, ik + base, res)

            res = lax.fori_loop(0, K, fb_k, jnp.zeros((SG, KP), jnp.int32))
            out_ref[0, 0, pl.ds(rg * SG, SG), :] = res

        return 0

    lax.fori_loop(0, NT // SG, fb_subgroup, 0)


def _run_topk(x_v):
    xvt = x_v.transpose(0, 2, 1)
    return pl.pallas_call(
        _topk_kernel,
        grid=(B, T),
        in_specs=[
            pl.BlockSpec((1, NT, 3), lambda b, t: (b, t, 0)),
            pl.BlockSpec((1, 3, N), lambda b, t: (b, 0, 0)),
        ],
        out_specs=pl.BlockSpec((1, 1, NT, KP), lambda b, t: (b, t, 0, 0)),
        out_shape=jax.ShapeDtypeStruct((B, T, NT, KP), jnp.int32),
        scratch_shapes=[
            pltpu.VMEM((NT, N), jnp.float32),
            pltpu.SMEM((NT // SG,), jnp.int32),
        ],
    )(x_v, xvt)


NW = 32
ROWS = B * N * K
PW = ROWS // NW
CH = 512
NCH = PW // CH


def _gather_body(table_hbm, idx_hbm, out_hbm, idx_v, rows_v, sem):
    wid = lax.axis_index("s") * 2 + lax.axis_index("c")
    base = wid * PW
    pltpu.sync_copy(idx_hbm.at[pl.ds(base, PW)], idx_v)

    def chunk(ci, _):
        pltpu.async_copy(
            table_hbm.at[idx_v.at[pl.ds(ci * CH, CH)]], rows_v, sem
        ).wait()
        pltpu.sync_copy(rows_v, out_hbm.at[pl.ds(base + ci * CH, CH)])
        return 0

    lax.fori_loop(0, NCH, chunk, 0)


@functools.cache
def _make_gather():
    return pl.kernel(
        _gather_body,
        mesh=plsc.VectorSubcoreMesh(core_axis_name="c", subcore_axis_name="s"),
        out_type=jax.ShapeDtypeStruct((ROWS, D), jnp.float32),
        scratch_types=[
            pltpu.VMEM((PW,), jnp.int32),
            pltpu.VMEM((CH, D), jnp.float32),
            pltpu.SemaphoreType.DMA,
        ],
    )


def _mm_kernel(flat_ref, wt_ref, b_ref, out_ref):
    out_ref[...] = (
        jnp.dot(flat_ref[...], wt_ref[...], preferred_element_type=jnp.float32)
        + b_ref[...]
    )


def _run_mm(flat, wt, bias):
    return pl.pallas_call(
        _mm_kernel,
        grid=(B * T,),
        in_specs=[
            pl.BlockSpec((NT, K * D), lambda i: (i, 0)),
            pl.BlockSpec((K * D, E), lambda i: (0, 0)),
            pl.BlockSpec((1, E), lambda i: (0, 0)),
        ],
        out_specs=pl.BlockSpec((NT, E), lambda i: (i, 0)),
        out_shape=jax.ShapeDtypeStruct((B * T * NT, E), jnp.float32),
    )(flat, wt, bias)


def kernel(x, x_v, W, b):
    idxg = _run_topk(x_v)
    idx = idxg[..., :K].reshape(ROWS)
    table = x.reshape(B * N, D)
    g = _make_gather()(table, idx)
    flat = g.reshape(B * T * NT, K * D)
    out = _run_mm(flat, W.T, b.reshape(1, E))
    return out.reshape(B, N, E)


# speedup vs baseline: 18.0434x; 1.2805x over previous
import functools

import jax
import jax.numpy as jnp
from jax import lax
from jax.experimental import pallas as pl
from jax.experimental.pallas import tpu as pltpu
from jax.experimental.pallas import tpu_sc as plsc

B, N, D, K, E = 4, 4096, 128, 27, 256
NT = 256
T = N // NT
SG = 16
KP = 32


S = 6
NC_ = N // 128


def _topk_kernel(xv_ref, xvt_ref, out_ref, dmat_ref, flags_ref):
    b = pl.program_id(0)
    xq = xv_ref[0]
    xpt = xvt_ref[0]
    d2q = (xq[:, 0:1] * xq[:, 0:1] + xq[:, 1:2] * xq[:, 1:2]) + xq[:, 2:3] * xq[:, 2:3]
    d2p = (xpt[0:1, :] * xpt[0:1, :] + xpt[1:2, :] * xpt[1:2, :]) + xpt[2:3, :] * xpt[2:3, :]
    dot = jnp.dot(xq, xpt, preferred_element_type=jnp.float32)
    dmat_ref[...] = (d2q + d2p) - 2.0 * dot

    base = b * N
    inf = jnp.float32(jnp.inf)
    bigi = jnp.int32(N)
    liota = lax.broadcasted_iota(jnp.int32, (SG, 128), 1)
    kiota = lax.broadcasted_iota(jnp.int32, (SG, KP), 1)
    kiota128 = liota

    def subgroup(rg):
        v = [jnp.full((SG, 128), inf, jnp.float32) for _ in range(S)]
        gi = [jnp.full((SG, 128), bigi, jnp.int32) for _ in range(S)]
        for c in range(NC_):
            t = dmat_ref[pl.ds(rg * SG, SG), pl.ds(c * 128, 128)]
            ti = liota + (c * 128)
            for j in range(S):
                lt = t < v[j]
                nv = jnp.where(lt, t, v[j])
                nt = jnp.where(lt, v[j], t)
                ni = jnp.where(lt, ti, gi[j])
                nti = jnp.where(lt, gi[j], ti)
                v[j], gi[j], t, ti = nv, ni, nt, nti
        res = jnp.zeros((SG, KP), jnp.int32)
        flag = jnp.zeros((SG, 128), jnp.int32)
        for k in range(K):
            m = jnp.min(v[0], axis=1, keepdims=True)
            ik = jnp.min(jnp.where(v[0] == m, gi[0], bigi), axis=1, keepdims=True)
            res = jnp.where(kiota == k, ik + base, res)
            ext = (v[0] == m) & (gi[0] == ik)
            for j in range(S - 1):
                v[j] = jnp.where(ext, v[j + 1], v[j])
                gi[j] = jnp.where(ext, gi[j + 1], gi[j])
            v[S - 1] = jnp.where(ext, inf, v[S - 1])
            gi[S - 1] = jnp.where(ext, bigi, gi[S - 1])
            flag = flag | jnp.where(ext & (v[0] == inf), 1, 0)
        out_ref[0, 0, pl.ds(rg * SG, SG), :] = res
        flags_ref[rg] = jnp.max(flag)

    for rg_ in range(NT // SG):
        subgroup(rg_)

    giota = lax.broadcasted_iota(jnp.int32, (SG, N), 1)

    def fb_subgroup(rg, _):
        @pl.when(flags_ref[rg] > 0)
        def _():
            def fb_k(k, res):
                d = dmat_ref[pl.ds(rg * SG, SG), :]
                m = jnp.min(d, axis=1, keepdims=True)
                ik = jnp.min(jnp.where(d == m, giota, bigi), axis=1, keepdims=True)
                dmat_ref[pl.ds(rg * SG, SG), :] = jnp.where(giota == ik, inf, d)
                return jnp.where(kiota == k, ik + base, res)

            res = lax.fori_loop(0, K, fb_k, jnp.zeros((SG, KP), jnp.int32))
            out_ref[0, 0, pl.ds(rg * SG, SG), :] = res

        return 0

    lax.fori_loop(0, NT // SG, fb_subgroup, 0)


def _run_topk(x_v):
    xvt = x_v.transpose(0, 2, 1)
    return pl.pallas_call(
        _topk_kernel,
        grid=(B, T),
        in_specs=[
            pl.BlockSpec((1, NT, 3), lambda b, t: (b, t, 0)),
            pl.BlockSpec((1, 3, N), lambda b, t: (b, 0, 0)),
        ],
        out_specs=pl.BlockSpec((1, 1, NT, KP), lambda b, t: (b, t, 0, 0)),
        out_shape=jax.ShapeDtypeStruct((B, T, NT, KP), jnp.int32),
        scratch_shapes=[
            pltpu.VMEM((NT, N), jnp.float32),
            pltpu.SMEM((NT // SG,), jnp.int32),
        ],
    )(x_v, xvt)


NW = 32
ROWS = B * N * K
PW = ROWS // NW
CH = 512
NCH = PW // CH


def _gather_body(table_hbm, idx_hbm, out_hbm, idx_v, rows_v, sem):
    wid = lax.axis_index("s") * 2 + lax.axis_index("c")
    base = wid * PW
    pltpu.sync_copy(idx_hbm.at[pl.ds(base, PW)], idx_v)

    def chunk(ci, _):
        pltpu.async_copy(
            table_hbm.at[idx_v.at[pl.ds(ci * CH, CH)]], rows_v, sem
        ).wait()
        pltpu.sync_copy(rows_v, out_hbm.at[pl.ds(base + ci * CH, CH)])
        return 0

    lax.fori_loop(0, NCH, chunk, 0)


@functools.cache
def _make_gather():
    return pl.kernel(
        _gather_body,
        mesh=plsc.VectorSubcoreMesh(core_axis_name="c", subcore_axis_name="s"),
        out_type=jax.ShapeDtypeStruct((ROWS, D), jnp.float32),
        scratch_types=[
            pltpu.VMEM((PW,), jnp.int32),
            pltpu.VMEM((CH, D), jnp.float32),
            pltpu.SemaphoreType.DMA,
        ],
    )


def _mm_kernel(flat_ref, wt_ref, b_ref, out_ref):
    out_ref[...] = (
        jnp.dot(flat_ref[...], wt_ref[...], preferred_element_type=jnp.float32)
        + b_ref[...]
    )


def _run_mm(flat, wt, bias):
    return pl.pallas_call(
        _mm_kernel,
        grid=(B * T,),
        in_specs=[
            pl.BlockSpec((NT, K * D), lambda i: (i, 0)),
            pl.BlockSpec((K * D, E), lambda i: (0, 0)),
            pl.BlockSpec((1, E), lambda i: (0, 0)),
        ],
        out_specs=pl.BlockSpec((NT, E), lambda i: (i, 0)),
        out_shape=jax.ShapeDtypeStruct((B * T * NT, E), jnp.float32),
    )(flat, wt, bias)


def kernel(x, x_v, W, b):
    idxg = _run_topk(x_v)
    idx = idxg[..., :K].reshape(ROWS)
    table = x.reshape(B * N, D)
    g = _make_gather()(table, idx)
    flat = g.reshape(B * T * NT, K * D)
    out = _run_mm(flat, W.T, b.reshape(1, E))
    return out.reshape(B, N, E)


# speedup vs baseline: 18.3722x; 1.0182x over previous
import functools

import jax
import jax.numpy as jnp
from jax import lax
from jax.experimental import pallas as pl
from jax.experimental.pallas import tpu as pltpu
from jax.experimental.pallas import tpu_sc as plsc

B, N, D, K, E = 4, 4096, 128, 27, 256
NT = 256
T = N // NT
SG = 16
KP = 32


S = 6
NC_ = N // 128


def _topk_kernel(xv_ref, xvt_ref, out_ref, dmat_ref, flags_ref):
    b = pl.program_id(0)
    xq = xv_ref[0]
    xpt = xvt_ref[0]
    d2p = (xpt[0:1, :] * xpt[0:1, :] + xpt[1:2, :] * xpt[1:2, :]) + xpt[2:3, :] * xpt[2:3, :]

    base = b * N
    inf = jnp.float32(jnp.inf)
    bigi = jnp.int32(N)
    liota = lax.broadcasted_iota(jnp.int32, (SG, 128), 1)
    kiota = lax.broadcasted_iota(jnp.int32, (SG, KP), 1)

    def subgroup(rg):
        xq_sg = xq[rg * SG:(rg + 1) * SG, :]
        d2q = (
            xq_sg[:, 0:1] * xq_sg[:, 0:1] + xq_sg[:, 1:2] * xq_sg[:, 1:2]
        ) + xq_sg[:, 2:3] * xq_sg[:, 2:3]
        dot = jnp.dot(xq_sg, xpt, preferred_element_type=jnp.float32)
        dmat = (d2q + d2p) - 2.0 * dot
        dmat_ref[pl.ds(rg * SG, SG), :] = dmat
        v = [jnp.full((SG, 128), inf, jnp.float32) for _ in range(S)]
        gi = [jnp.full((SG, 128), bigi, jnp.int32) for _ in range(S)]
        for c in range(NC_):
            t = dmat[:, c * 128:(c + 1) * 128]
            ti = liota + (c * 128)
            for j in range(S):
                lt = t < v[j]
                nv = jnp.where(lt, t, v[j])
                ni = jnp.where(lt, ti, gi[j])
                if j < S - 1:
                    t = jnp.where(lt, v[j], t)
                    ti = jnp.where(lt, gi[j], ti)
                v[j], gi[j] = nv, ni
        res = jnp.zeros((SG, KP), jnp.int32)
        flag = jnp.zeros((SG, 128), jnp.int32)
        for k in range(K):
            m = jnp.min(v[0], axis=1, keepdims=True)
            ik = jnp.min(jnp.where(v[0] == m, gi[0], bigi), axis=1, keepdims=True)
            res = jnp.where(kiota == k, ik + base, res)
            ext = (v[0] == m) & (gi[0] == ik)
            for j in range(S - 1):
                v[j] = jnp.where(ext, v[j + 1], v[j])
                gi[j] = jnp.where(ext, gi[j + 1], gi[j])
            v[S - 1] = jnp.where(ext, inf, v[S - 1])
            gi[S - 1] = jnp.where(ext, bigi, gi[S - 1])
            flag = flag | jnp.where(ext & (v[0] == inf), 1, 0)
        out_ref[0, 0, pl.ds(rg * SG, SG), :] = res
        flags_ref[rg] = jnp.max(flag)

    for rg_ in range(NT // SG):
        subgroup(rg_)

    giota = lax.broadcasted_iota(jnp.int32, (SG, N), 1)

    def fb_subgroup(rg, _):
        @pl.when(flags_ref[rg] > 0)
        def _():
            def fb_k(k, res):
                d = dmat_ref[pl.ds(rg * SG, SG), :]
                m = jnp.min(d, axis=1, keepdims=True)
                ik = jnp.min(jnp.where(d == m, giota, bigi), axis=1, keepdims=True)
                dmat_ref[pl.ds(rg * SG, SG), :] = jnp.where(giota == ik, inf, d)
                return jnp.where(kiota == k, ik + base, res)

            res = lax.fori_loop(0, K, fb_k, jnp.zeros((SG, KP), jnp.int32))
            out_ref[0, 0, pl.ds(rg * SG, SG), :] = res

        return 0

    lax.fori_loop(0, NT // SG, fb_subgroup, 0)


def _run_topk(x_v):
    xvt = x_v.transpose(0, 2, 1)
    return pl.pallas_call(
        _topk_kernel,
        grid=(B, T),
        in_specs=[
            pl.BlockSpec((1, NT, 3), lambda b, t: (b, t, 0)),
            pl.BlockSpec((1, 3, N), lambda b, t: (b, 0, 0)),
        ],
        out_specs=pl.BlockSpec((1, 1, NT, KP), lambda b, t: (b, t, 0, 0)),
        out_shape=jax.ShapeDtypeStruct((B, T, NT, KP), jnp.int32),
        scratch_shapes=[
            pltpu.VMEM((NT, N), jnp.float32),
            pltpu.SMEM((NT // SG,), jnp.int32),
        ],
    )(x_v, xvt)


NW = 32
ROWS = B * N * K
PW = ROWS // NW
CH = 512
NCH = PW // CH


def _gather_body(table_hbm, idx_hbm, out_hbm, idx_v, rows_v, sem):
    wid = lax.axis_index("s") * 2 + lax.axis_index("c")
    base = wid * PW
    pltpu.sync_copy(idx_hbm.at[pl.ds(base, PW)], idx_v)

    def chunk(ci, _):
        pltpu.async_copy(
            table_hbm.at[idx_v.at[pl.ds(ci * CH, CH)]], rows_v, sem
        ).wait()
        pltpu.sync_copy(rows_v, out_hbm.at[pl.ds(base + ci * CH, CH)])
        return 0

    lax.fori_loop(0, NCH, chunk, 0)


@functools.cache
def _make_gather():
    return pl.kernel(
        _gather_body,
        mesh=plsc.VectorSubcoreMesh(core_axis_name="c", subcore_axis_name="s"),
        out_type=jax.ShapeDtypeStruct((ROWS, D), jnp.float32),
        scratch_types=[
            pltpu.VMEM((PW,), jnp.int32),
            pltpu.VMEM((CH, D), jnp.float32),
            pltpu.SemaphoreType.DMA,
        ],
    )


def _mm_kernel(flat_ref, wt_ref, b_ref, out_ref):
    out_ref[...] = (
        jnp.dot(flat_ref[...], wt_ref[...], preferred_element_type=jnp.float32)
        + b_ref[...]
    )


def _run_mm(flat, wt, bias):
    return pl.pallas_call(
        _mm_kernel,
        grid=(B * T,),
        in_specs=[
            pl.BlockSpec((NT, K * D), lambda i: (i, 0)),
            pl.BlockSpec((K * D, E), lambda i: (0, 0)),
            pl.BlockSpec((1, E), lambda i: (0, 0)),
        ],
        out_specs=pl.BlockSpec((NT, E), lambda i: (i, 0)),
        out_shape=jax.ShapeDtypeStruct((B * T * NT, E), jnp.float32),
    )(flat, wt, bias)


def kernel(x, x_v, W, b):
    idxg = _run_topk(x_v)
    idx = idxg[..., :K].reshape(ROWS)
    table = x.reshape(B * N, D)
    g = _make_gather()(table, idx)
    flat = g.reshape(B * T * NT, K * D)
    out = _run_mm(flat, W.T, b.reshape(1, E))
    return out.reshape(B, N, E)
